# Initial kernel scaffold; baseline (speedup 1.0000x reference)
import jax
import jax.numpy as jnp
from jax.experimental import pallas as pl


def kernel(activations, enc, dec, expert_b, router_b, router):
    raise NotImplementedError("write your pallas kernel here")


# speedup vs baseline: 1.1328x; 1.1328x over previous
import functools

import jax
import jax.numpy as jnp
from jax import lax
from jax.experimental import pallas as pl
from jax.experimental.pallas import tpu as pltpu

H = 2048
E = 16
NF = 16384
FE = NF // E

BF = 256
GRID = FE // BF


def _router_body(act_ref, rb_ref, router_ref, idx_ref, maxp_ref):
    x = act_ref[...] - rb_ref[...]
    logits = jnp.dot(x, router_ref[...],
                     preferred_element_type=jnp.float32)
    m = jnp.max(logits)
    s = jnp.sum(jnp.exp(logits - m))
    iota = lax.broadcasted_iota(jnp.int32, (1, E), 1)
    idx = jnp.min(jnp.where(logits == m, iota, E))
    idx_ref[0] = idx
    maxp_ref[0] = 1.0 / s


def _main_body(idx_ref, act_ref, eb_ref, maxp_ref, enc_ref, dec_ref, out_ref):
    i = pl.program_id(0)
    x = act_ref[...] - eb_ref[...]
    f = jnp.dot(x, enc_ref[0], preferred_element_type=jnp.float32)
    f = jnp.maximum(f, 0.0)
    contrib = jnp.dot(f, dec_ref[0], preferred_element_type=jnp.float32)

    @pl.when(i == 0)
    def _init():
        out_ref[...] = contrib

    @pl.when(i > 0)
    def _acc():
        out_ref[...] += contrib

    @pl.when(i == GRID - 1)
    def _fin():
        out_ref[...] = out_ref[...] * maxp_ref[0] + eb_ref[...]


def kernel(activations, enc, dec, expert_b, router_b, router):
    act2 = activations.reshape(1, H)
    rb2 = router_b.reshape(1, H)
    eb2 = expert_b.reshape(1, H)

    idx, maxp = pl.pallas_call(
        _router_body,
        out_shape=[
            jax.ShapeDtypeStruct((1,), jnp.int32),
            jax.ShapeDtypeStruct((1,), jnp.float32),
        ],
        in_specs=[
            pl.BlockSpec(memory_space=pltpu.VMEM),
            pl.BlockSpec(memory_space=pltpu.VMEM),
            pl.BlockSpec(memory_space=pltpu.VMEM),
        ],
        out_specs=[
            pl.BlockSpec(memory_space=pltpu.SMEM),
            pl.BlockSpec(memory_space=pltpu.SMEM),
        ],
    )(act2, rb2, router)

    out = pl.pallas_call(
        _main_body,
        grid_spec=pltpu.PrefetchScalarGridSpec(
            num_scalar_prefetch=1,
            grid=(GRID,),
            in_specs=[
                pl.BlockSpec((1, H), lambda i, idx_ref: (0, 0)),
                pl.BlockSpec((1, H), lambda i, idx_ref: (0, 0)),
                pl.BlockSpec(memory_space=pltpu.SMEM),
                pl.BlockSpec((1, H, BF),
                             lambda i, idx_ref: (idx_ref[0], 0, i)),
                pl.BlockSpec((1, BF, H),
                             lambda i, idx_ref: (idx_ref[0], i, 0)),
            ],
            out_specs=pl.BlockSpec((1, H), lambda i, idx_ref: (0, 0)),
        ),
        out_shape=jax.ShapeDtypeStruct((1, H), jnp.float32),
    )(idx, act2, eb2, maxp, enc, dec)

    return out.reshape(H)
